# baseline (speedup 1.0000x reference)
import functools

import jax
import jax.numpy as jnp
from jax import lax
from jax.experimental import pallas as pl
from jax.experimental.pallas import tpu as pltpu
from jax.experimental.pallas import tpu_sc as plsc

_BR = 128
_HALO = 8


def _roll(x, shift, axis):
    return pltpu.roll(x, shift % x.shape[axis], axis)


def _combine(bd, bv, cd, cv, use_max):
    better = (cd > bd) if use_max else (cd < bd)
    return jnp.where(better, cd, bd), jnp.where(better, cv, bv)


def _map_body(a_ext, c_ext, o_ref, row0, h, w):
    br = o_ref.shape[0]
    d_ext = jnp.abs(a_ext - c_ext)

    row_rel = jax.lax.broadcasted_iota(jnp.int32, (br, w), 0)
    col = jax.lax.broadcasted_iota(jnp.int32, (br, w), 1)
    even_r = (row_rel % 2) == 0
    even_c = (col % 2) == 0

    d0, d1, d2 = (d_ext[k:k + br] for k in range(3))
    c0, c1, c2 = (c_ext[k:k + br] for k in range(3))
    md, mv = _combine(d0, c0, d1, c1, False)
    md, mv = _combine(md, mv, d2, c2, False)
    xd, xv = _combine(d0, c0, d1, c1, True)
    xd, xv = _combine(xd, xv, d2, c2, True)

    md1, mv1 = _roll(md, -1, 1), _roll(mv, -1, 1)
    md2, mv2 = _roll(md, -2, 1), _roll(mv, -2, 1)
    xd1, xv1 = _roll(xd, -1, 1), _roll(xv, -1, 1)
    xd2, xv2 = _roll(xd, -2, 1), _roll(xv, -2, 1)
    md, mv = _combine(md, mv, md1, mv1, False)
    md, mv = _combine(md, mv, md2, mv2, False)
    xd, xv = _combine(xd, xv, xd1, xv1, True)
    xd, xv = _combine(xd, xv, xd2, xv2, True)

    out = mv + xv

    out = jnp.where(even_c, out, _roll(out, 1, 1))
    out = jnp.where(even_r, out, _roll(out, 1, 0))

    row_g = row_rel + row0
    out = jnp.where(col == w - 2, _roll(out, 2, 1), out)
    out = jnp.where(row_g == h - 2, _roll(out, 2, 0), out)
    out = jnp.where((row_g == h - 1) | (col == w - 1), 2.0 * c_ext[0:br], out)
    o_ref[...] = out


def _dc_kernel(a_ref, p_ref, n_ref, ah_ref, ph_ref, nh_ref, po_ref, no_ref,
               *, h, w, br):
    b = pl.program_id(0)
    row0 = b * br
    a_ext = jnp.concatenate([a_ref[...], ah_ref[...]], axis=0)
    p_ext = jnp.concatenate([p_ref[...], ph_ref[...]], axis=0)
    n_ext = jnp.concatenate([n_ref[...], nh_ref[...]], axis=0)
    _map_body(a_ext, p_ext, po_ref, row0, h, w)
    _map_body(a_ext, n_ext, no_ref, row0, h, w)


_SC_NC, _SC_NS, _SC_L = 2, 16, 16
_NW = _SC_NC * _SC_NS
_CH = 4


def _sc_body(a_hbm, p_hbm, n_hbm, pos_hbm, neg_hbm, a_v, p_v, n_v,
             pos_v, neg_v, *, h, w):
    wid = lax.axis_index("s") * _SC_NC + lax.axis_index("c")
    nwr_tile = (h // 2) // _NW
    nchunks = nwr_tile // _CH
    ngroups = (w // 2) // _SC_L
    iota = lax.iota(jnp.int32, _SC_L)
    lane_last = iota == (_SC_L - 1)

    def chunk_body(chunk, chunk_carry):
        i0 = wid * nwr_tile + chunk * _CH
        start = 2 * i0
        halo = jnp.minimum(start + 8, h - 8)
        pltpu.sync_copy(a_hbm.at[pl.ds(start * w, 8 * w)], a_v.at[pl.ds(0, 8 * w)])
        pltpu.sync_copy(p_hbm.at[pl.ds(start * w, 8 * w)], p_v.at[pl.ds(0, 8 * w)])
        pltpu.sync_copy(n_hbm.at[pl.ds(start * w, 8 * w)], n_v.at[pl.ds(0, 8 * w)])
        pltpu.sync_copy(a_hbm.at[pl.ds(halo * w, w)], a_v.at[pl.ds(8 * w, w)])
        pltpu.sync_copy(p_hbm.at[pl.ds(halo * w, w)], p_v.at[pl.ds(8 * w, w)])
        pltpu.sync_copy(n_hbm.at[pl.ds(halo * w, w)], n_v.at[pl.ds(8 * w, w)])

        for ir in range(_CH):
            iw = i0 + ir
            base = jnp.minimum(2 * iw, h - 4) - start

            def group_body(g, carry, base=base, ir=ir):
                jv = g * _SC_L + iota
                cb = jnp.minimum(2 * jv, w - 4)
                pdm = pvm = pdx = pvx = None
                ndm = nvm = ndx = nvx = None
                for k in range(3):
                    rbase = jnp.broadcast_to((base + k) * w, (_SC_L,))
                    for l in range(3):
                        cvec = rbase + cb + l
                        av = plsc.load_gather(a_v, [cvec])
                        pv = plsc.load_gather(p_v, [cvec])
                        nv = plsc.load_gather(n_v, [cvec])
                        dp = jnp.abs(av - pv)
                        dn = jnp.abs(av - nv)
                        if pdm is None:
                            pdm = pdx = dp
                            pvm = pvx = pv
                            ndm = ndx = dn
                            nvm = nvx = nv
                        else:
                            m = dp < pdm
                            pdm = jnp.where(m, dp, pdm)
                            pvm = jnp.where(m, pv, pvm)
                            m = dp > pdx
                            pdx = jnp.where(m, dp, pdx)
                            pvx = jnp.where(m, pv, pvx)
                            m = dn < ndm
                            ndm = jnp.where(m, dn, ndm)
                            nvm = jnp.where(m, nv, nvm)
                            m = dn > ndx
                            ndx = jnp.where(m, dn, ndx)
                            nvx = jnp.where(m, nv, nvx)
                vp = pvm + pvx
                vn = nvm + nvx
                oc0 = 2 * jv + (2 * ir) * w
                oc1 = oc0 + 1
                oc2 = 2 * jv + (2 * ir + 1) * w
                oc3 = oc2 + 1
                plsc.store_scatter(pos_v, [oc0], vp)
                plsc.store_scatter(pos_v, [oc1], vp)
                plsc.store_scatter(pos_v, [oc2], vp)
                plsc.store_scatter(pos_v, [oc3], vp)
                plsc.store_scatter(neg_v, [oc0], vn)
                plsc.store_scatter(neg_v, [oc1], vn)
                plsc.store_scatter(neg_v, [oc2], vn)
                plsc.store_scatter(neg_v, [oc3], vn)
                return carry

            lax.fori_loop(0, ngroups, group_body, 0)

        for orel in range(2 * _CH):
            tail = pl.ds(orel * w + w - _SC_L, _SC_L)
            for c_v, o_v in ((p_v, pos_v), (n_v, neg_v)):
                o_v[tail] = jnp.where(lane_last, 2.0 * c_v[tail], o_v[tail])

        @pl.when(i0 + _CH == h // 2)
        def _fix_last_row():
            for cc in range(w // _SC_L):
                osl = pl.ds((2 * _CH - 1) * w + cc * _SC_L, _SC_L)
                csl = pl.ds(7 * w + cc * _SC_L, _SC_L)
                pos_v[osl] = 2.0 * p_v[csl]
                neg_v[osl] = 2.0 * n_v[csl]

        pltpu.sync_copy(pos_v, pos_hbm.at[pl.ds(2 * i0 * w, 2 * _CH * w)])
        pltpu.sync_copy(neg_v, neg_hbm.at[pl.ds(2 * i0 * w, 2 * _CH * w)])
        return chunk_carry

    lax.fori_loop(0, nchunks, chunk_body, 0)


def _sc_kernel(anchor, positive, negative):
    h, w = anchor.shape
    mesh = plsc.VectorSubcoreMesh(core_axis_name="c", subcore_axis_name="s",
                                  num_cores=_SC_NC, num_subcores=_SC_NS)
    body = functools.partial(_sc_body, h=h, w=w)
    pos, neg = pl.kernel(
        body,
        out_type=(jax.ShapeDtypeStruct((h * w,), jnp.float32),
                  jax.ShapeDtypeStruct((h * w,), jnp.float32)),
        mesh=mesh,
        scratch_types=[
            pltpu.VMEM((9 * w,), jnp.float32),
            pltpu.VMEM((9 * w,), jnp.float32),
            pltpu.VMEM((9 * w,), jnp.float32),
            pltpu.VMEM((2 * _CH * w,), jnp.float32),
            pltpu.VMEM((2 * _CH * w,), jnp.float32),
        ],
        compiler_params=pltpu.CompilerParams(needs_layout_passes=False),
    )(anchor.reshape(-1), positive.reshape(-1), negative.reshape(-1))
    return pos.reshape(h, w), neg.reshape(h, w)


def kernel(anchor, positive, negative):
    return _sc_kernel(anchor, positive, negative)


def _tc_kernel(anchor, positive, negative):
    h, w = anchor.shape
    br = min(_BR, h)
    nb = h // br
    halo_blocks = h // _HALO

    def main_spec():
        return pl.BlockSpec((br, w), lambda b: (b, 0))

    def halo_spec():
        return pl.BlockSpec(
            (_HALO, w),
            lambda b: (jnp.minimum((b + 1) * (br // _HALO), halo_blocks - 1), 0),
        )

    body = functools.partial(_dc_kernel, h=h, w=w, br=br)
    pos, neg = pl.pallas_call(
        body,
        grid=(nb,),
        in_specs=[main_spec(), main_spec(), main_spec(),
                  halo_spec(), halo_spec(), halo_spec()],
        out_specs=[pl.BlockSpec((br, w), lambda b: (b, 0))] * 2,
        out_shape=[jax.ShapeDtypeStruct((h, w), jnp.float32)] * 2,
        compiler_params=pltpu.CompilerParams(
            dimension_semantics=("arbitrary",),
        ),
    )(anchor, positive, negative, anchor, positive, negative)
    return (pos, neg)


# speedup vs baseline: 1.2332x; 1.2332x over previous
import functools

import jax
import jax.numpy as jnp
from jax import lax
from jax.experimental import pallas as pl
from jax.experimental.pallas import tpu as pltpu
from jax.experimental.pallas import tpu_sc as plsc

_BR = 128
_HALO = 8


def _roll(x, shift, axis):
    return pltpu.roll(x, shift % x.shape[axis], axis)


def _combine(bd, bv, cd, cv, use_max):
    better = (cd > bd) if use_max else (cd < bd)
    return jnp.where(better, cd, bd), jnp.where(better, cv, bv)


def _map_body(a_ext, c_ext, o_ref, row0, h, w):
    br = o_ref.shape[0]
    d_ext = jnp.abs(a_ext - c_ext)

    row_rel = jax.lax.broadcasted_iota(jnp.int32, (br, w), 0)
    col = jax.lax.broadcasted_iota(jnp.int32, (br, w), 1)
    even_r = (row_rel % 2) == 0
    even_c = (col % 2) == 0

    d0, d1, d2 = (d_ext[k:k + br] for k in range(3))
    c0, c1, c2 = (c_ext[k:k + br] for k in range(3))
    md, mv = _combine(d0, c0, d1, c1, False)
    md, mv = _combine(md, mv, d2, c2, False)
    xd, xv = _combine(d0, c0, d1, c1, True)
    xd, xv = _combine(xd, xv, d2, c2, True)

    md1, mv1 = _roll(md, -1, 1), _roll(mv, -1, 1)
    md2, mv2 = _roll(md, -2, 1), _roll(mv, -2, 1)
    xd1, xv1 = _roll(xd, -1, 1), _roll(xv, -1, 1)
    xd2, xv2 = _roll(xd, -2, 1), _roll(xv, -2, 1)
    md, mv = _combine(md, mv, md1, mv1, False)
    md, mv = _combine(md, mv, md2, mv2, False)
    xd, xv = _combine(xd, xv, xd1, xv1, True)
    xd, xv = _combine(xd, xv, xd2, xv2, True)

    out = mv + xv

    out = jnp.where(even_c, out, _roll(out, 1, 1))
    out = jnp.where(even_r, out, _roll(out, 1, 0))

    row_g = row_rel + row0
    out = jnp.where(col == w - 2, _roll(out, 2, 1), out)
    out = jnp.where(row_g == h - 2, _roll(out, 2, 0), out)
    out = jnp.where((row_g == h - 1) | (col == w - 1), 2.0 * c_ext[0:br], out)
    o_ref[...] = out


def _dc_kernel(a_ref, p_ref, n_ref, ah_ref, ph_ref, nh_ref, po_ref, no_ref,
               *, h, w, br):
    b = pl.program_id(0)
    row0 = b * br
    a_ext = jnp.concatenate([a_ref[...], ah_ref[...]], axis=0)
    p_ext = jnp.concatenate([p_ref[...], ph_ref[...]], axis=0)
    n_ext = jnp.concatenate([n_ref[...], nh_ref[...]], axis=0)
    _map_body(a_ext, p_ext, po_ref, row0, h, w)
    _map_body(a_ext, n_ext, no_ref, row0, h, w)


_SC_NC, _SC_NS, _SC_L = 2, 16, 16
_NW = _SC_NC * _SC_NS
_CH = 4


def _sc_body(a_hbm, p_hbm, n_hbm, pos_hbm, neg_hbm, a_v, p_v, n_v,
             pos_v, neg_v, *, h, w, row0):
    wid = lax.axis_index("s") * _SC_NC + lax.axis_index("c")
    nwr_tile = ((h - row0) // 2) // _NW
    nchunks = nwr_tile // _CH
    ngroups = (w // 2) // _SC_L
    iota = lax.iota(jnp.int32, _SC_L)
    lane_last = iota == (_SC_L - 1)

    def chunk_body(chunk, chunk_carry):
        i0 = row0 // 2 + wid * nwr_tile + chunk * _CH
        start = 2 * i0
        halo = jnp.minimum(start + 8, h - 8)
        pltpu.sync_copy(a_hbm.at[pl.ds(start * w, 8 * w)], a_v.at[pl.ds(0, 8 * w)])
        pltpu.sync_copy(p_hbm.at[pl.ds(start * w, 8 * w)], p_v.at[pl.ds(0, 8 * w)])
        pltpu.sync_copy(n_hbm.at[pl.ds(start * w, 8 * w)], n_v.at[pl.ds(0, 8 * w)])
        pltpu.sync_copy(a_hbm.at[pl.ds(halo * w, w)], a_v.at[pl.ds(8 * w, w)])
        pltpu.sync_copy(p_hbm.at[pl.ds(halo * w, w)], p_v.at[pl.ds(8 * w, w)])
        pltpu.sync_copy(n_hbm.at[pl.ds(halo * w, w)], n_v.at[pl.ds(8 * w, w)])

        for ir in range(_CH):
            iw = i0 + ir
            base = jnp.minimum(2 * iw, h - 4) - start

            def group_body(g, carry, base=base, ir=ir):
                jv = g * _SC_L + iota
                cb = jnp.minimum(2 * jv, w - 4)
                pdm = pvm = pdx = pvx = None
                ndm = nvm = ndx = nvx = None
                for k in range(3):
                    rbase = jnp.broadcast_to((base + k) * w, (_SC_L,))
                    for l in range(3):
                        cvec = rbase + cb + l
                        av = plsc.load_gather(a_v, [cvec])
                        pv = plsc.load_gather(p_v, [cvec])
                        nv = plsc.load_gather(n_v, [cvec])
                        dp = jnp.abs(av - pv)
                        dn = jnp.abs(av - nv)
                        if pdm is None:
                            pdm = pdx = dp
                            pvm = pvx = pv
                            ndm = ndx = dn
                            nvm = nvx = nv
                        else:
                            m = dp < pdm
                            pdm = jnp.where(m, dp, pdm)
                            pvm = jnp.where(m, pv, pvm)
                            m = dp > pdx
                            pdx = jnp.where(m, dp, pdx)
                            pvx = jnp.where(m, pv, pvx)
                            m = dn < ndm
                            ndm = jnp.where(m, dn, ndm)
                            nvm = jnp.where(m, nv, nvm)
                            m = dn > ndx
                            ndx = jnp.where(m, dn, ndx)
                            nvx = jnp.where(m, nv, nvx)
                vp = pvm + pvx
                vn = nvm + nvx
                oc0 = 2 * jv + (2 * ir) * w
                oc1 = oc0 + 1
                oc2 = 2 * jv + (2 * ir + 1) * w
                oc3 = oc2 + 1
                plsc.store_scatter(pos_v, [oc0], vp)
                plsc.store_scatter(pos_v, [oc1], vp)
                plsc.store_scatter(pos_v, [oc2], vp)
                plsc.store_scatter(pos_v, [oc3], vp)
                plsc.store_scatter(neg_v, [oc0], vn)
                plsc.store_scatter(neg_v, [oc1], vn)
                plsc.store_scatter(neg_v, [oc2], vn)
                plsc.store_scatter(neg_v, [oc3], vn)
                return carry

            lax.fori_loop(0, ngroups, group_body, 0)

        for orel in range(2 * _CH):
            tail = pl.ds(orel * w + w - _SC_L, _SC_L)
            for c_v, o_v in ((p_v, pos_v), (n_v, neg_v)):
                o_v[tail] = jnp.where(lane_last, 2.0 * c_v[tail], o_v[tail])

        @pl.when(i0 + _CH == h // 2)
        def _fix_last_row():
            for cc in range(w // _SC_L):
                osl = pl.ds((2 * _CH - 1) * w + cc * _SC_L, _SC_L)
                csl = pl.ds(7 * w + cc * _SC_L, _SC_L)
                pos_v[osl] = 2.0 * p_v[csl]
                neg_v[osl] = 2.0 * n_v[csl]

        obase = (2 * i0 - row0) * w
        pltpu.sync_copy(pos_v, pos_hbm.at[pl.ds(obase, 2 * _CH * w)])
        pltpu.sync_copy(neg_v, neg_hbm.at[pl.ds(obase, 2 * _CH * w)])
        return chunk_carry

    lax.fori_loop(0, nchunks, chunk_body, 0)


def _sc_kernel(anchor, positive, negative, row0=0):
    h, w = anchor.shape
    mesh = plsc.VectorSubcoreMesh(core_axis_name="c", subcore_axis_name="s",
                                  num_cores=_SC_NC, num_subcores=_SC_NS)
    body = functools.partial(_sc_body, h=h, w=w, row0=row0)
    orows = h - row0
    pos, neg = pl.kernel(
        body,
        out_type=(jax.ShapeDtypeStruct((orows * w,), jnp.float32),
                  jax.ShapeDtypeStruct((orows * w,), jnp.float32)),
        mesh=mesh,
        scratch_types=[
            pltpu.VMEM((9 * w,), jnp.float32),
            pltpu.VMEM((9 * w,), jnp.float32),
            pltpu.VMEM((9 * w,), jnp.float32),
            pltpu.VMEM((2 * _CH * w,), jnp.float32),
            pltpu.VMEM((2 * _CH * w,), jnp.float32),
        ],
        compiler_params=pltpu.CompilerParams(needs_layout_passes=False),
    )(anchor.reshape(-1), positive.reshape(-1), negative.reshape(-1))
    return pos.reshape(orows, w), neg.reshape(orows, w)


def kernel(anchor, positive, negative):
    tc_pos, tc_neg = _tc_kernel(anchor, positive, negative, out_rows=_TC_ROWS)
    sc_pos, sc_neg = _sc_kernel(anchor, positive, negative, row0=_TC_ROWS)
    return (jnp.concatenate([tc_pos, sc_pos], axis=0),
            jnp.concatenate([tc_neg, sc_neg], axis=0))


_TC_ROWS = 1280


def _tc_kernel(anchor, positive, negative, out_rows=None):
    h, w = anchor.shape
    br = min(_BR, h)
    if out_rows is None:
        out_rows = h
    nb = out_rows // br
    halo_blocks = h // _HALO

    def main_spec():
        return pl.BlockSpec((br, w), lambda b: (b, 0))

    def halo_spec():
        return pl.BlockSpec(
            (_HALO, w),
            lambda b: (jnp.minimum((b + 1) * (br // _HALO), halo_blocks - 1), 0),
        )

    body = functools.partial(_dc_kernel, h=h, w=w, br=br)
    pos, neg = pl.pallas_call(
        body,
        grid=(nb,),
        in_specs=[main_spec(), main_spec(), main_spec(),
                  halo_spec(), halo_spec(), halo_spec()],
        out_specs=[pl.BlockSpec((br, w), lambda b: (b, 0))] * 2,
        out_shape=[jax.ShapeDtypeStruct((out_rows, w), jnp.float32)] * 2,
        compiler_params=pltpu.CompilerParams(
            dimension_semantics=("arbitrary",),
        ),
    )(anchor, positive, negative, anchor, positive, negative)
    return (pos, neg)


# speedup vs baseline: 1.7614x; 1.4283x over previous
import functools

import jax
import jax.numpy as jnp
from jax import lax
from jax.experimental import pallas as pl
from jax.experimental.pallas import tpu as pltpu
from jax.experimental.pallas import tpu_sc as plsc

_BR = 128
_HALO = 8


def _roll(x, shift, axis):
    return pltpu.roll(x, shift % x.shape[axis], axis)


def _combine(bd, bv, cd, cv, use_max):
    better = (cd > bd) if use_max else (cd < bd)
    return jnp.where(better, cd, bd), jnp.where(better, cv, bv)


def _map_body(a_ext, c_ext, o_ref, row0, h, w):
    br = o_ref.shape[0]
    d_ext = jnp.abs(a_ext - c_ext)

    row_rel = jax.lax.broadcasted_iota(jnp.int32, (br, w), 0)
    col = jax.lax.broadcasted_iota(jnp.int32, (br, w), 1)
    even_r = (row_rel % 2) == 0
    even_c = (col % 2) == 0

    d0, d1, d2 = (d_ext[k:k + br] for k in range(3))
    c0, c1, c2 = (c_ext[k:k + br] for k in range(3))
    md, mv = _combine(d0, c0, d1, c1, False)
    md, mv = _combine(md, mv, d2, c2, False)
    xd, xv = _combine(d0, c0, d1, c1, True)
    xd, xv = _combine(xd, xv, d2, c2, True)

    md1, mv1 = _roll(md, -1, 1), _roll(mv, -1, 1)
    md2, mv2 = _roll(md, -2, 1), _roll(mv, -2, 1)
    xd1, xv1 = _roll(xd, -1, 1), _roll(xv, -1, 1)
    xd2, xv2 = _roll(xd, -2, 1), _roll(xv, -2, 1)
    md, mv = _combine(md, mv, md1, mv1, False)
    md, mv = _combine(md, mv, md2, mv2, False)
    xd, xv = _combine(xd, xv, xd1, xv1, True)
    xd, xv = _combine(xd, xv, xd2, xv2, True)

    out = mv + xv

    out = jnp.where(even_c, out, _roll(out, 1, 1))
    out = jnp.where(even_r, out, _roll(out, 1, 0))

    row_g = row_rel + row0
    out = jnp.where(col == w - 2, _roll(out, 2, 1), out)
    out = jnp.where(row_g == h - 2, _roll(out, 2, 0), out)
    out = jnp.where((row_g == h - 1) | (col == w - 1), 2.0 * c_ext[0:br], out)
    o_ref[...] = out


def _dc_kernel(a_ref, p_ref, n_ref, ah_ref, ph_ref, nh_ref, po_ref, no_ref,
               *, h, w, br):
    b = pl.program_id(0)
    row0 = b * br
    a_ext = jnp.concatenate([a_ref[...], ah_ref[...]], axis=0)
    p_ext = jnp.concatenate([p_ref[...], ph_ref[...]], axis=0)
    n_ext = jnp.concatenate([n_ref[...], nh_ref[...]], axis=0)
    _map_body(a_ext, p_ext, po_ref, row0, h, w)
    _map_body(a_ext, n_ext, no_ref, row0, h, w)


_SC_NC, _SC_NS, _SC_L = 2, 16, 16
_NW = _SC_NC * _SC_NS
_CH = 4


def _sc_body(a_hbm, p_hbm, n_hbm, pos_hbm, neg_hbm, a_v, p_v, n_v,
             pos_v, neg_v, *, h, w, row0):
    wid = lax.axis_index("s") * _SC_NC + lax.axis_index("c")
    nwr_tile = ((h - row0) // 2) // _NW
    nchunks = nwr_tile // _CH
    ngroups = (w // 2) // _SC_L
    iota = lax.iota(jnp.int32, _SC_L)
    lane_last = iota == (_SC_L - 1)

    def chunk_body(chunk, chunk_carry):
        i0 = row0 // 2 + wid * nwr_tile + chunk * _CH
        start = 2 * i0
        halo = jnp.minimum(start + 8, h - 8)
        for hbm, vbuf in ((a_hbm, a_v), (p_hbm, p_v), (n_hbm, n_v)):
            for r in range(8):
                pltpu.sync_copy(hbm.at[start + r], vbuf.at[pl.ds(r * w, w)])
            pltpu.sync_copy(hbm.at[halo], vbuf.at[pl.ds(8 * w, w)])

        for ir in range(_CH):
            iw = i0 + ir
            base = jnp.minimum(2 * iw, h - 4) - start

            def group_body(g, carry, base=base, ir=ir):
                jv = g * _SC_L + iota
                cb = jnp.minimum(2 * jv, w - 4)
                pdm = pvm = pdx = pvx = None
                ndm = nvm = ndx = nvx = None
                for k in range(3):
                    rbase = jnp.broadcast_to((base + k) * w, (_SC_L,))
                    for l in range(3):
                        cvec = rbase + cb + l
                        av = plsc.load_gather(a_v, [cvec])
                        pv = plsc.load_gather(p_v, [cvec])
                        nv = plsc.load_gather(n_v, [cvec])
                        dp = jnp.abs(av - pv)
                        dn = jnp.abs(av - nv)
                        if pdm is None:
                            pdm = pdx = dp
                            pvm = pvx = pv
                            ndm = ndx = dn
                            nvm = nvx = nv
                        else:
                            m = dp < pdm
                            pdm = jnp.where(m, dp, pdm)
                            pvm = jnp.where(m, pv, pvm)
                            m = dp > pdx
                            pdx = jnp.where(m, dp, pdx)
                            pvx = jnp.where(m, pv, pvx)
                            m = dn < ndm
                            ndm = jnp.where(m, dn, ndm)
                            nvm = jnp.where(m, nv, nvm)
                            m = dn > ndx
                            ndx = jnp.where(m, dn, ndx)
                            nvx = jnp.where(m, nv, nvx)
                vp = pvm + pvx
                vn = nvm + nvx
                oc0 = 2 * jv + (2 * ir) * w
                oc1 = oc0 + 1
                oc2 = 2 * jv + (2 * ir + 1) * w
                oc3 = oc2 + 1
                plsc.store_scatter(pos_v, [oc0], vp)
                plsc.store_scatter(pos_v, [oc1], vp)
                plsc.store_scatter(pos_v, [oc2], vp)
                plsc.store_scatter(pos_v, [oc3], vp)
                plsc.store_scatter(neg_v, [oc0], vn)
                plsc.store_scatter(neg_v, [oc1], vn)
                plsc.store_scatter(neg_v, [oc2], vn)
                plsc.store_scatter(neg_v, [oc3], vn)
                return carry

            lax.fori_loop(0, ngroups, group_body, 0)

        for orel in range(2 * _CH):
            tail = pl.ds(orel * w + w - _SC_L, _SC_L)
            for c_v, o_v in ((p_v, pos_v), (n_v, neg_v)):
                o_v[tail] = jnp.where(lane_last, 2.0 * c_v[tail], o_v[tail])

        @pl.when(i0 + _CH == h // 2)
        def _fix_last_row():
            for cc in range(w // _SC_L):
                osl = pl.ds((2 * _CH - 1) * w + cc * _SC_L, _SC_L)
                csl = pl.ds(7 * w + cc * _SC_L, _SC_L)
                pos_v[osl] = 2.0 * p_v[csl]
                neg_v[osl] = 2.0 * n_v[csl]

        obase = 2 * i0 - row0
        for vbuf, hbm in ((pos_v, pos_hbm), (neg_v, neg_hbm)):
            for r in range(2 * _CH):
                pltpu.sync_copy(vbuf.at[pl.ds(r * w, w)], hbm.at[obase + r])
        return chunk_carry

    lax.fori_loop(0, nchunks, chunk_body, 0)


def _sc_kernel(anchor, positive, negative, row0=0):
    h, w = anchor.shape
    mesh = plsc.VectorSubcoreMesh(core_axis_name="c", subcore_axis_name="s",
                                  num_cores=_SC_NC, num_subcores=_SC_NS)
    body = functools.partial(_sc_body, h=h, w=w, row0=row0)
    orows = h - row0
    pos, neg = pl.kernel(
        body,
        out_type=(jax.ShapeDtypeStruct((orows, w), jnp.float32),
                  jax.ShapeDtypeStruct((orows, w), jnp.float32)),
        mesh=mesh,
        scratch_types=[
            pltpu.VMEM((9 * w,), jnp.float32),
            pltpu.VMEM((9 * w,), jnp.float32),
            pltpu.VMEM((9 * w,), jnp.float32),
            pltpu.VMEM((2 * _CH * w,), jnp.float32),
            pltpu.VMEM((2 * _CH * w,), jnp.float32),
        ],
        compiler_params=pltpu.CompilerParams(needs_layout_passes=False),
    )(anchor, positive, negative)
    return pos, neg


def kernel(anchor, positive, negative):
    tc_pos, tc_neg = _tc_kernel(anchor, positive, negative, out_rows=_TC_ROWS)
    sc_pos, sc_neg = _sc_kernel(anchor, positive, negative, row0=_TC_ROWS)
    return (jnp.concatenate([tc_pos, sc_pos], axis=0),
            jnp.concatenate([tc_neg, sc_neg], axis=0))


_TC_ROWS = 1280


def _tc_kernel(anchor, positive, negative, out_rows=None):
    h, w = anchor.shape
    br = min(_BR, h)
    if out_rows is None:
        out_rows = h
    nb = out_rows // br
    halo_blocks = h // _HALO

    def main_spec():
        return pl.BlockSpec((br, w), lambda b: (b, 0))

    def halo_spec():
        return pl.BlockSpec(
            (_HALO, w),
            lambda b: (jnp.minimum((b + 1) * (br // _HALO), halo_blocks - 1), 0),
        )

    body = functools.partial(_dc_kernel, h=h, w=w, br=br)
    pos, neg = pl.pallas_call(
        body,
        grid=(nb,),
        in_specs=[main_spec(), main_spec(), main_spec(),
                  halo_spec(), halo_spec(), halo_spec()],
        out_specs=[pl.BlockSpec((br, w), lambda b: (b, 0))] * 2,
        out_shape=[jax.ShapeDtypeStruct((out_rows, w), jnp.float32)] * 2,
        compiler_params=pltpu.CompilerParams(
            dimension_semantics=("arbitrary",),
        ),
    )(anchor, positive, negative, anchor, positive, negative)
    return (pos, neg)


# speedup vs baseline: 1.9213x; 1.0908x over previous
import functools

import jax
import jax.numpy as jnp
from jax import lax
from jax.experimental import pallas as pl
from jax.experimental.pallas import tpu as pltpu
from jax.experimental.pallas import tpu_sc as plsc

_BR = 128
_HALO = 8


def _roll(x, shift, axis):
    return pltpu.roll(x, shift % x.shape[axis], axis)


def _combine(bd, bv, cd, cv, use_max):
    better = (cd > bd) if use_max else (cd < bd)
    return jnp.where(better, cd, bd), jnp.where(better, cv, bv)


def _map_body(a_ext, c_ext, o_ref, row0, h, w):
    br = o_ref.shape[0]
    d_ext = jnp.abs(a_ext - c_ext)

    row_rel = jax.lax.broadcasted_iota(jnp.int32, (br, w), 0)
    col = jax.lax.broadcasted_iota(jnp.int32, (br, w), 1)
    even_r = (row_rel % 2) == 0
    even_c = (col % 2) == 0

    d0, d1, d2 = (d_ext[k:k + br] for k in range(3))
    c0, c1, c2 = (c_ext[k:k + br] for k in range(3))
    md, mv = _combine(d0, c0, d1, c1, False)
    md, mv = _combine(md, mv, d2, c2, False)
    xd, xv = _combine(d0, c0, d1, c1, True)
    xd, xv = _combine(xd, xv, d2, c2, True)

    md1, mv1 = _roll(md, -1, 1), _roll(mv, -1, 1)
    md2, mv2 = _roll(md, -2, 1), _roll(mv, -2, 1)
    xd1, xv1 = _roll(xd, -1, 1), _roll(xv, -1, 1)
    xd2, xv2 = _roll(xd, -2, 1), _roll(xv, -2, 1)
    md, mv = _combine(md, mv, md1, mv1, False)
    md, mv = _combine(md, mv, md2, mv2, False)
    xd, xv = _combine(xd, xv, xd1, xv1, True)
    xd, xv = _combine(xd, xv, xd2, xv2, True)

    out = mv + xv

    out = jnp.where(even_c, out, _roll(out, 1, 1))
    out = jnp.where(even_r, out, _roll(out, 1, 0))

    row_g = row_rel + row0
    out = jnp.where(col == w - 2, _roll(out, 2, 1), out)
    out = jnp.where(row_g == h - 2, _roll(out, 2, 0), out)
    out = jnp.where((row_g == h - 1) | (col == w - 1), 2.0 * c_ext[0:br], out)
    o_ref[...] = out


def _dc_kernel(a_ref, p_ref, n_ref, ah_ref, ph_ref, nh_ref, po_ref, no_ref,
               *, h, w, br):
    b = pl.program_id(0)
    row0 = b * br
    a_ext = jnp.concatenate([a_ref[...], ah_ref[...]], axis=0)
    p_ext = jnp.concatenate([p_ref[...], ph_ref[...]], axis=0)
    n_ext = jnp.concatenate([n_ref[...], nh_ref[...]], axis=0)
    _map_body(a_ext, p_ext, po_ref, row0, h, w)
    _map_body(a_ext, n_ext, no_ref, row0, h, w)


_SC_NC, _SC_NS, _SC_L = 2, 16, 16
_NW = _SC_NC * _SC_NS
_CH = 4


def _sc_body(a_hbm, p_hbm, n_hbm, pos_hbm, neg_hbm, a_v, p_v, n_v,
             pos_v, neg_v, *, h, w, row0):
    wid = lax.axis_index("s") * _SC_NC + lax.axis_index("c")
    nwr_tile = ((h - row0) // 2) // _NW
    nchunks = nwr_tile // _CH
    ngroups = (w // 2) // _SC_L
    iota = lax.iota(jnp.int32, _SC_L)
    lane_last = iota == (_SC_L - 1)

    def chunk_body(chunk, chunk_carry):
        i0 = row0 // 2 + wid * nwr_tile + chunk * _CH
        start = 2 * i0
        halo = jnp.minimum(start + 8, h - 8)
        for hbm, vbuf in ((a_hbm, a_v), (p_hbm, p_v), (n_hbm, n_v)):
            for r in range(8):
                pltpu.sync_copy(hbm.at[start + r], vbuf.at[pl.ds(r * w, w)])
            pltpu.sync_copy(hbm.at[halo], vbuf.at[pl.ds(8 * w, w)])

        for ir in range(_CH):
            iw = i0 + ir
            base = jnp.minimum(2 * iw, h - 4) - start

            def group_body(g, carry, base=base, ir=ir):
                jv = g * _SC_L + iota
                cb = jnp.minimum(2 * jv, w - 4)
                pdm = pvm = pdx = pvx = None
                ndm = nvm = ndx = nvx = None
                for k in range(3):
                    rbase = jnp.broadcast_to((base + k) * w, (_SC_L,))
                    for l in range(3):
                        cvec = rbase + cb + l
                        av = plsc.load_gather(a_v, [cvec])
                        pv = plsc.load_gather(p_v, [cvec])
                        nv = plsc.load_gather(n_v, [cvec])
                        dp = jnp.abs(av - pv)
                        dn = jnp.abs(av - nv)
                        if pdm is None:
                            pdm = pdx = dp
                            pvm = pvx = pv
                            ndm = ndx = dn
                            nvm = nvx = nv
                        else:
                            m = dp < pdm
                            pdm = jnp.where(m, dp, pdm)
                            pvm = jnp.where(m, pv, pvm)
                            m = dp > pdx
                            pdx = jnp.where(m, dp, pdx)
                            pvx = jnp.where(m, pv, pvx)
                            m = dn < ndm
                            ndm = jnp.where(m, dn, ndm)
                            nvm = jnp.where(m, nv, nvm)
                            m = dn > ndx
                            ndx = jnp.where(m, dn, ndx)
                            nvx = jnp.where(m, nv, nvx)
                vp = pvm + pvx
                vn = nvm + nvx
                oc0 = 2 * jv + (2 * ir) * w
                oc1 = oc0 + 1
                oc2 = 2 * jv + (2 * ir + 1) * w
                oc3 = oc2 + 1
                plsc.store_scatter(pos_v, [oc0], vp)
                plsc.store_scatter(pos_v, [oc1], vp)
                plsc.store_scatter(pos_v, [oc2], vp)
                plsc.store_scatter(pos_v, [oc3], vp)
                plsc.store_scatter(neg_v, [oc0], vn)
                plsc.store_scatter(neg_v, [oc1], vn)
                plsc.store_scatter(neg_v, [oc2], vn)
                plsc.store_scatter(neg_v, [oc3], vn)
                return carry

            lax.fori_loop(0, ngroups, group_body, 0)

        for orel in range(2 * _CH):
            tail = pl.ds(orel * w + w - _SC_L, _SC_L)
            for c_v, o_v in ((p_v, pos_v), (n_v, neg_v)):
                o_v[tail] = jnp.where(lane_last, 2.0 * c_v[tail], o_v[tail])

        @pl.when(i0 + _CH == h // 2)
        def _fix_last_row():
            for cc in range(w // _SC_L):
                osl = pl.ds((2 * _CH - 1) * w + cc * _SC_L, _SC_L)
                csl = pl.ds(7 * w + cc * _SC_L, _SC_L)
                pos_v[osl] = 2.0 * p_v[csl]
                neg_v[osl] = 2.0 * n_v[csl]

        obase = 2 * i0 - row0
        for vbuf, hbm in ((pos_v, pos_hbm), (neg_v, neg_hbm)):
            for r in range(2 * _CH):
                pltpu.sync_copy(vbuf.at[pl.ds(r * w, w)], hbm.at[obase + r])
        return chunk_carry

    lax.fori_loop(0, nchunks, chunk_body, 0)


def _sc_kernel(anchor, positive, negative, row0=0):
    h, w = anchor.shape
    mesh = plsc.VectorSubcoreMesh(core_axis_name="c", subcore_axis_name="s",
                                  num_cores=_SC_NC, num_subcores=_SC_NS)
    body = functools.partial(_sc_body, h=h, w=w, row0=row0)
    orows = h - row0
    pos, neg = pl.kernel(
        body,
        out_type=(jax.ShapeDtypeStruct((orows, w), jnp.float32),
                  jax.ShapeDtypeStruct((orows, w), jnp.float32)),
        mesh=mesh,
        scratch_types=[
            pltpu.VMEM((9 * w,), jnp.float32),
            pltpu.VMEM((9 * w,), jnp.float32),
            pltpu.VMEM((9 * w,), jnp.float32),
            pltpu.VMEM((2 * _CH * w,), jnp.float32),
            pltpu.VMEM((2 * _CH * w,), jnp.float32),
        ],
        compiler_params=pltpu.CompilerParams(needs_layout_passes=False),
    )(anchor, positive, negative)
    return pos, neg


def kernel(anchor, positive, negative):
    tc_pos, tc_neg = _tc_kernel(anchor, positive, negative, out_rows=_TC_ROWS)
    sc_pos, sc_neg = _sc_kernel(anchor, positive, negative, row0=_TC_ROWS)
    return (lax.dynamic_update_slice(tc_pos, sc_pos, (_TC_ROWS, 0)),
            lax.dynamic_update_slice(tc_neg, sc_neg, (_TC_ROWS, 0)))


_TC_ROWS = 1280


def _tc_kernel(anchor, positive, negative, out_rows=None):
    h, w = anchor.shape
    br = min(_BR, h)
    if out_rows is None:
        out_rows = h
    nb = out_rows // br
    halo_blocks = h // _HALO

    def main_spec():
        return pl.BlockSpec((br, w), lambda b: (b, 0))

    def halo_spec():
        return pl.BlockSpec(
            (_HALO, w),
            lambda b: (jnp.minimum((b + 1) * (br // _HALO), halo_blocks - 1), 0),
        )

    body = functools.partial(_dc_kernel, h=h, w=w, br=br)
    pos, neg = pl.pallas_call(
        body,
        grid=(nb,),
        in_specs=[main_spec(), main_spec(), main_spec(),
                  halo_spec(), halo_spec(), halo_spec()],
        out_specs=[pl.BlockSpec((br, w), lambda b: (b, 0))] * 2,
        out_shape=[jax.ShapeDtypeStruct((h, w), jnp.float32)] * 2,
        compiler_params=pltpu.CompilerParams(
            dimension_semantics=("arbitrary",),
        ),
    )(anchor, positive, negative, anchor, positive, negative)
    return (pos, neg)


# speedup vs baseline: 2.0281x; 1.0556x over previous
import functools

import jax
import jax.numpy as jnp
from jax import lax
from jax.experimental import pallas as pl
from jax.experimental.pallas import tpu as pltpu
from jax.experimental.pallas import tpu_sc as plsc

_BR = 128
_HALO = 8


def _roll(x, shift, axis):
    return pltpu.roll(x, shift % x.shape[axis], axis)


def _combine(bd, bv, cd, cv, use_max):
    better = (cd > bd) if use_max else (cd < bd)
    return jnp.where(better, cd, bd), jnp.where(better, cv, bv)


def _map_body(a_ext, c_ext, o_ref, row0, h, w):
    br = o_ref.shape[0]
    d_ext = jnp.abs(a_ext - c_ext)

    row_rel = jax.lax.broadcasted_iota(jnp.int32, (br, w), 0)
    col = jax.lax.broadcasted_iota(jnp.int32, (br, w), 1)
    even_r = (row_rel % 2) == 0
    even_c = (col % 2) == 0

    d0, d1, d2 = (d_ext[k:k + br] for k in range(3))
    c0, c1, c2 = (c_ext[k:k + br] for k in range(3))
    md, mv = _combine(d0, c0, d1, c1, False)
    md, mv = _combine(md, mv, d2, c2, False)
    xd, xv = _combine(d0, c0, d1, c1, True)
    xd, xv = _combine(xd, xv, d2, c2, True)

    md1, mv1 = _roll(md, -1, 1), _roll(mv, -1, 1)
    md2, mv2 = _roll(md, -2, 1), _roll(mv, -2, 1)
    xd1, xv1 = _roll(xd, -1, 1), _roll(xv, -1, 1)
    xd2, xv2 = _roll(xd, -2, 1), _roll(xv, -2, 1)
    md, mv = _combine(md, mv, md1, mv1, False)
    md, mv = _combine(md, mv, md2, mv2, False)
    xd, xv = _combine(xd, xv, xd1, xv1, True)
    xd, xv = _combine(xd, xv, xd2, xv2, True)

    out = mv + xv

    out = jnp.where(even_c, out, _roll(out, 1, 1))
    out = jnp.where(even_r, out, _roll(out, 1, 0))

    row_g = row_rel + row0
    out = jnp.where(col == w - 2, _roll(out, 2, 1), out)
    out = jnp.where(row_g == h - 2, _roll(out, 2, 0), out)
    out = jnp.where((row_g == h - 1) | (col == w - 1), 2.0 * c_ext[0:br], out)
    o_ref[...] = out


def _dc_kernel(a_ref, p_ref, n_ref, ah_ref, ph_ref, nh_ref, po_ref, no_ref,
               *, h, w, br):
    b = pl.program_id(0)
    row0 = b * br
    a_ext = jnp.concatenate([a_ref[...], ah_ref[...]], axis=0)
    p_ext = jnp.concatenate([p_ref[...], ph_ref[...]], axis=0)
    n_ext = jnp.concatenate([n_ref[...], nh_ref[...]], axis=0)
    _map_body(a_ext, p_ext, po_ref, row0, h, w)
    _map_body(a_ext, n_ext, no_ref, row0, h, w)


_SC_NC, _SC_NS, _SC_L = 2, 16, 16
_NW = _SC_NC * _SC_NS
_CH = 4


def _sc_body(a_hbm, p_hbm, n_hbm, pos_hbm, neg_hbm, a_v, p_v, n_v,
             pos_v, neg_v, in_sem, out_sem, *, h, w, row0):
    wid = lax.axis_index("s") * _SC_NC + lax.axis_index("c")
    nwr_tile = ((h - row0) // 2) // _NW
    nchunks = nwr_tile // _CH
    ngroups = (w // 2) // _SC_L
    iota = lax.iota(jnp.int32, _SC_L)
    lane_last = iota == (_SC_L - 1)

    out_cps = []
    for chunk in range(nchunks):
        i0 = row0 // 2 + wid * nwr_tile + chunk * _CH
        start = 2 * i0
        halo = jnp.minimum(start + 8, h - 8)
        in_cps = []
        for hbm, vbuf in ((a_hbm, a_v), (p_hbm, p_v), (n_hbm, n_v)):
            for r in range(8):
                in_cps.append(pltpu.async_copy(
                    hbm.at[start + r], vbuf.at[pl.ds(r * w, w)], in_sem))
            in_cps.append(pltpu.async_copy(
                hbm.at[halo], vbuf.at[pl.ds(8 * w, w)], in_sem))
        for cp in out_cps:
            cp.wait()
        out_cps = []
        for cp in in_cps:
            cp.wait()

        for ir in range(_CH):
            iw = i0 + ir
            base = jnp.minimum(2 * iw, h - 4) - start

            def group_body(g, carry, base=base, ir=ir):
                jv = g * _SC_L + iota
                cb = jnp.minimum(2 * jv, w - 4)
                pdm = pvm = pdx = pvx = None
                ndm = nvm = ndx = nvx = None
                for k in range(3):
                    rbase = jnp.broadcast_to((base + k) * w, (_SC_L,))
                    for l in range(3):
                        cvec = rbase + cb + l
                        av = plsc.load_gather(a_v, [cvec])
                        pv = plsc.load_gather(p_v, [cvec])
                        nv = plsc.load_gather(n_v, [cvec])
                        dp = jnp.abs(av - pv)
                        dn = jnp.abs(av - nv)
                        if pdm is None:
                            pdm = pdx = dp
                            pvm = pvx = pv
                            ndm = ndx = dn
                            nvm = nvx = nv
                        else:
                            m = dp < pdm
                            pdm = jnp.where(m, dp, pdm)
                            pvm = jnp.where(m, pv, pvm)
                            m = dp > pdx
                            pdx = jnp.where(m, dp, pdx)
                            pvx = jnp.where(m, pv, pvx)
                            m = dn < ndm
                            ndm = jnp.where(m, dn, ndm)
                            nvm = jnp.where(m, nv, nvm)
                            m = dn > ndx
                            ndx = jnp.where(m, dn, ndx)
                            nvx = jnp.where(m, nv, nvx)
                vp = pvm + pvx
                vn = nvm + nvx
                oc0 = 2 * jv + (2 * ir) * w
                oc1 = oc0 + 1
                oc2 = 2 * jv + (2 * ir + 1) * w
                oc3 = oc2 + 1
                plsc.store_scatter(pos_v, [oc0], vp)
                plsc.store_scatter(pos_v, [oc1], vp)
                plsc.store_scatter(pos_v, [oc2], vp)
                plsc.store_scatter(pos_v, [oc3], vp)
                plsc.store_scatter(neg_v, [oc0], vn)
                plsc.store_scatter(neg_v, [oc1], vn)
                plsc.store_scatter(neg_v, [oc2], vn)
                plsc.store_scatter(neg_v, [oc3], vn)
                return carry

            lax.fori_loop(0, ngroups, group_body, 0)

        for orel in range(2 * _CH):
            tail = pl.ds(orel * w + w - _SC_L, _SC_L)
            for c_v, o_v in ((p_v, pos_v), (n_v, neg_v)):
                o_v[tail] = jnp.where(lane_last, 2.0 * c_v[tail], o_v[tail])

        @pl.when(i0 + _CH == h // 2)
        def _fix_last_row():
            for cc in range(w // _SC_L):
                osl = pl.ds((2 * _CH - 1) * w + cc * _SC_L, _SC_L)
                csl = pl.ds(7 * w + cc * _SC_L, _SC_L)
                pos_v[osl] = 2.0 * p_v[csl]
                neg_v[osl] = 2.0 * n_v[csl]

        obase = 2 * i0 - row0
        for vbuf, hbm in ((pos_v, pos_hbm), (neg_v, neg_hbm)):
            for r in range(2 * _CH):
                out_cps.append(pltpu.async_copy(
                    vbuf.at[pl.ds(r * w, w)], hbm.at[obase + r], out_sem))
    for cp in out_cps:
        cp.wait()


def _sc_kernel(anchor, positive, negative, row0=0):
    h, w = anchor.shape
    mesh = plsc.VectorSubcoreMesh(core_axis_name="c", subcore_axis_name="s",
                                  num_cores=_SC_NC, num_subcores=_SC_NS)
    body = functools.partial(_sc_body, h=h, w=w, row0=row0)
    orows = h - row0
    pos, neg = pl.kernel(
        body,
        out_type=(jax.ShapeDtypeStruct((orows, w), jnp.float32),
                  jax.ShapeDtypeStruct((orows, w), jnp.float32)),
        mesh=mesh,
        scratch_types=[
            pltpu.VMEM((9 * w,), jnp.float32),
            pltpu.VMEM((9 * w,), jnp.float32),
            pltpu.VMEM((9 * w,), jnp.float32),
            pltpu.VMEM((2 * _CH * w,), jnp.float32),
            pltpu.VMEM((2 * _CH * w,), jnp.float32),
            pltpu.SemaphoreType.DMA,
            pltpu.SemaphoreType.DMA,
        ],
        compiler_params=pltpu.CompilerParams(needs_layout_passes=False),
    )(anchor, positive, negative)
    return pos, neg


def kernel(anchor, positive, negative):
    tc_pos, tc_neg = _tc_kernel(anchor, positive, negative, out_rows=_TC_ROWS)
    sc_pos, sc_neg = _sc_kernel(anchor, positive, negative, row0=_TC_ROWS)
    return (lax.dynamic_update_slice(tc_pos, sc_pos, (_TC_ROWS, 0)),
            lax.dynamic_update_slice(tc_neg, sc_neg, (_TC_ROWS, 0)))


_TC_ROWS = 1280


def _tc_kernel(anchor, positive, negative, out_rows=None):
    h, w = anchor.shape
    br = min(_BR, h)
    if out_rows is None:
        out_rows = h
    nb = out_rows // br
    halo_blocks = h // _HALO

    def main_spec():
        return pl.BlockSpec((br, w), lambda b: (b, 0))

    def halo_spec():
        return pl.BlockSpec(
            (_HALO, w),
            lambda b: (jnp.minimum((b + 1) * (br // _HALO), halo_blocks - 1), 0),
        )

    body = functools.partial(_dc_kernel, h=h, w=w, br=br)
    pos, neg = pl.pallas_call(
        body,
        grid=(nb,),
        in_specs=[main_spec(), main_spec(), main_spec(),
                  halo_spec(), halo_spec(), halo_spec()],
        out_specs=[pl.BlockSpec((br, w), lambda b: (b, 0))] * 2,
        out_shape=[jax.ShapeDtypeStruct((h, w), jnp.float32)] * 2,
        compiler_params=pltpu.CompilerParams(
            dimension_semantics=("arbitrary",),
        ),
    )(anchor, positive, negative, anchor, positive, negative)
    return (pos, neg)


# speedup vs baseline: 2.3720x; 1.1696x over previous
import functools

import jax
import jax.numpy as jnp
from jax import lax
from jax.experimental import pallas as pl
from jax.experimental.pallas import tpu as pltpu
from jax.experimental.pallas import tpu_sc as plsc

_BR = 128
_HALO = 8


def _roll(x, shift, axis):
    return pltpu.roll(x, shift % x.shape[axis], axis)


def _combine(bd, bv, cd, cv, use_max):
    better = (cd > bd) if use_max else (cd < bd)
    return jnp.where(better, cd, bd), jnp.where(better, cv, bv)


def _map_body(a_ext, c_ext, o_ref, row0, h, w):
    br = o_ref.shape[0]
    d_ext = jnp.abs(a_ext - c_ext)

    row_rel = jax.lax.broadcasted_iota(jnp.int32, (br, w), 0)
    col = jax.lax.broadcasted_iota(jnp.int32, (br, w), 1)
    even_r = (row_rel % 2) == 0
    even_c = (col % 2) == 0

    d0, d1, d2 = (d_ext[k:k + br] for k in range(3))
    c0, c1, c2 = (c_ext[k:k + br] for k in range(3))
    md, mv = _combine(d0, c0, d1, c1, False)
    md, mv = _combine(md, mv, d2, c2, False)
    xd, xv = _combine(d0, c0, d1, c1, True)
    xd, xv = _combine(xd, xv, d2, c2, True)

    md1, mv1 = _roll(md, -1, 1), _roll(mv, -1, 1)
    md2, mv2 = _roll(md, -2, 1), _roll(mv, -2, 1)
    xd1, xv1 = _roll(xd, -1, 1), _roll(xv, -1, 1)
    xd2, xv2 = _roll(xd, -2, 1), _roll(xv, -2, 1)
    md, mv = _combine(md, mv, md1, mv1, False)
    md, mv = _combine(md, mv, md2, mv2, False)
    xd, xv = _combine(xd, xv, xd1, xv1, True)
    xd, xv = _combine(xd, xv, xd2, xv2, True)

    out = mv + xv

    out = jnp.where(even_c, out, _roll(out, 1, 1))
    out = jnp.where(even_r, out, _roll(out, 1, 0))

    row_g = row_rel + row0
    out = jnp.where(col == w - 2, _roll(out, 2, 1), out)
    out = jnp.where(row_g == h - 2, _roll(out, 2, 0), out)
    out = jnp.where((row_g == h - 1) | (col == w - 1), 2.0 * c_ext[0:br], out)
    o_ref[...] = out


def _dc_kernel(a_ref, p_ref, n_ref, ah_ref, ph_ref, nh_ref, po_ref, no_ref,
               *, h, w, br):
    b = pl.program_id(0)
    row0 = b * br
    a_ext = jnp.concatenate([a_ref[...], ah_ref[...]], axis=0)
    p_ext = jnp.concatenate([p_ref[...], ph_ref[...]], axis=0)
    n_ext = jnp.concatenate([n_ref[...], nh_ref[...]], axis=0)
    _map_body(a_ext, p_ext, po_ref, row0, h, w)
    _map_body(a_ext, n_ext, no_ref, row0, h, w)


_SC_NC, _SC_NS, _SC_L = 2, 16, 16
_NW = _SC_NC * _SC_NS
_CH = 4


def _sc_body(a_hbm, p_hbm, n_hbm, pos_hbm, neg_hbm, a_v, p_v, n_v,
             pos_v, neg_v, in_sem, out_sem, *, h, w, row0):
    wid = lax.axis_index("s") * _SC_NC + lax.axis_index("c")
    nwr_tile = ((h - row0) // 2) // _NW
    nchunks = nwr_tile // _CH
    ngroups = (w // 2) // _SC_L
    iota = lax.iota(jnp.int32, _SC_L)
    lane_last = iota == (_SC_L - 1)

    out_cps = []
    for chunk in range(nchunks):
        i0 = row0 // 2 + wid * nwr_tile + chunk * _CH
        start = 2 * i0
        halo = jnp.minimum(start + 8, h - 8)
        in_cps = []
        for hbm, vbuf in ((a_hbm, a_v), (p_hbm, p_v), (n_hbm, n_v)):
            for r in range(8):
                in_cps.append(pltpu.async_copy(
                    hbm.at[start + r], vbuf.at[pl.ds(r * w, w)], in_sem))
            in_cps.append(pltpu.async_copy(
                hbm.at[halo], vbuf.at[pl.ds(8 * w, w)], in_sem))
        for cp in out_cps:
            cp.wait()
        out_cps = []
        for cp in in_cps:
            cp.wait()

        for ir in range(_CH):
            iw = i0 + ir
            base = jnp.minimum(2 * iw, h - 4) - start

            def group_body(g, carry, base=base, ir=ir):
                jv = g * _SC_L + iota
                cb = jnp.minimum(2 * jv, w - 4)
                pdm = pvm = pdx = pvx = None
                ndm = nvm = ndx = nvx = None
                for k in range(3):
                    rbase = jnp.broadcast_to((base + k) * w, (_SC_L,))
                    for l in range(3):
                        cvec = rbase + cb + l
                        av = plsc.load_gather(a_v, [cvec])
                        pv = plsc.load_gather(p_v, [cvec])
                        nv = plsc.load_gather(n_v, [cvec])
                        dp = jnp.abs(av - pv)
                        dn = jnp.abs(av - nv)
                        if pdm is None:
                            pdm = pdx = dp
                            pvm = pvx = pv
                            ndm = ndx = dn
                            nvm = nvx = nv
                        else:
                            m = dp < pdm
                            pdm = jnp.where(m, dp, pdm)
                            pvm = jnp.where(m, pv, pvm)
                            m = dp > pdx
                            pdx = jnp.where(m, dp, pdx)
                            pvx = jnp.where(m, pv, pvx)
                            m = dn < ndm
                            ndm = jnp.where(m, dn, ndm)
                            nvm = jnp.where(m, nv, nvm)
                            m = dn > ndx
                            ndx = jnp.where(m, dn, ndx)
                            nvx = jnp.where(m, nv, nvx)
                vp = pvm + pvx
                vn = nvm + nvx
                oc0 = 2 * jv + (2 * ir) * w
                oc1 = oc0 + 1
                oc2 = 2 * jv + (2 * ir + 1) * w
                oc3 = oc2 + 1
                plsc.store_scatter(pos_v, [oc0], vp)
                plsc.store_scatter(pos_v, [oc1], vp)
                plsc.store_scatter(pos_v, [oc2], vp)
                plsc.store_scatter(pos_v, [oc3], vp)
                plsc.store_scatter(neg_v, [oc0], vn)
                plsc.store_scatter(neg_v, [oc1], vn)
                plsc.store_scatter(neg_v, [oc2], vn)
                plsc.store_scatter(neg_v, [oc3], vn)
                return carry

            lax.fori_loop(0, ngroups, group_body, 0)

        for orel in range(2 * _CH):
            tail = pl.ds(orel * w + w - _SC_L, _SC_L)
            for c_v, o_v in ((p_v, pos_v), (n_v, neg_v)):
                o_v[tail] = jnp.where(lane_last, 2.0 * c_v[tail], o_v[tail])

        @pl.when(i0 + _CH == h // 2)
        def _fix_last_row():
            for cc in range(w // _SC_L):
                osl = pl.ds((2 * _CH - 1) * w + cc * _SC_L, _SC_L)
                csl = pl.ds(7 * w + cc * _SC_L, _SC_L)
                pos_v[osl] = 2.0 * p_v[csl]
                neg_v[osl] = 2.0 * n_v[csl]

        obase = 2 * i0 - row0
        for vbuf, hbm in ((pos_v, pos_hbm), (neg_v, neg_hbm)):
            for r in range(2 * _CH):
                out_cps.append(pltpu.async_copy(
                    vbuf.at[pl.ds(r * w, w)], hbm.at[obase + r], out_sem))
    for cp in out_cps:
        cp.wait()


def _sc_kernel(anchor, positive, negative, row0=0):
    h, w = anchor.shape
    mesh = plsc.VectorSubcoreMesh(core_axis_name="c", subcore_axis_name="s",
                                  num_cores=_SC_NC, num_subcores=_SC_NS)
    body = functools.partial(_sc_body, h=h, w=w, row0=row0)
    orows = h - row0
    pos, neg = pl.kernel(
        body,
        out_type=(jax.ShapeDtypeStruct((orows, w), jnp.float32),
                  jax.ShapeDtypeStruct((orows, w), jnp.float32)),
        mesh=mesh,
        scratch_types=[
            pltpu.VMEM((9 * w,), jnp.float32),
            pltpu.VMEM((9 * w,), jnp.float32),
            pltpu.VMEM((9 * w,), jnp.float32),
            pltpu.VMEM((2 * _CH * w,), jnp.float32),
            pltpu.VMEM((2 * _CH * w,), jnp.float32),
            pltpu.SemaphoreType.DMA,
            pltpu.SemaphoreType.DMA,
        ],
        compiler_params=pltpu.CompilerParams(needs_layout_passes=False),
    )(anchor, positive, negative)
    return pos, neg


def kernel(anchor, positive, negative):
    tc_pos, tc_neg = _tc_kernel(anchor, positive, negative, out_rows=_TC_ROWS)
    sc_pos, sc_neg = _sc_kernel(anchor, positive, negative, row0=_TC_ROWS)
    return (lax.dynamic_update_slice(tc_pos, sc_pos, (_TC_ROWS, 0)),
            lax.dynamic_update_slice(tc_neg, sc_neg, (_TC_ROWS, 0)))


_TC_ROWS = 1024


def _tc_kernel(anchor, positive, negative, out_rows=None):
    h, w = anchor.shape
    br = min(_BR, h)
    if out_rows is None:
        out_rows = h
    nb = out_rows // br
    halo_blocks = h // _HALO

    def main_spec():
        return pl.BlockSpec((br, w), lambda b: (b, 0))

    def halo_spec():
        return pl.BlockSpec(
            (_HALO, w),
            lambda b: (jnp.minimum((b + 1) * (br // _HALO), halo_blocks - 1), 0),
        )

    body = functools.partial(_dc_kernel, h=h, w=w, br=br)
    pos, neg = pl.pallas_call(
        body,
        grid=(nb,),
        in_specs=[main_spec(), main_spec(), main_spec(),
                  halo_spec(), halo_spec(), halo_spec()],
        out_specs=[pl.BlockSpec((br, w), lambda b: (b, 0))] * 2,
        out_shape=[jax.ShapeDtypeStruct((h, w), jnp.float32)] * 2,
        compiler_params=pltpu.CompilerParams(
            dimension_semantics=("arbitrary",),
        ),
    )(anchor, positive, negative, anchor, positive, negative)
    return (pos, neg)


# speedup vs baseline: 2.4604x; 1.0372x over previous
import functools

import jax
import jax.numpy as jnp
from jax import lax
from jax.experimental import pallas as pl
from jax.experimental.pallas import tpu as pltpu
from jax.experimental.pallas import tpu_sc as plsc

_BR = 128
_HALO = 8


def _roll(x, shift, axis):
    return pltpu.roll(x, shift % x.shape[axis], axis)


def _combine(bd, bv, cd, cv, use_max):
    better = (cd > bd) if use_max else (cd < bd)
    return jnp.where(better, cd, bd), jnp.where(better, cv, bv)


def _map_body(a_ext, c_ext, o_ref, row0, h, w):
    br = o_ref.shape[0]
    d_ext = jnp.abs(a_ext - c_ext)

    row_rel = jax.lax.broadcasted_iota(jnp.int32, (br, w), 0)
    col = jax.lax.broadcasted_iota(jnp.int32, (br, w), 1)
    even_r = (row_rel % 2) == 0
    even_c = (col % 2) == 0

    d1, c1 = _roll(d_ext, -1, 1), _roll(c_ext, -1, 1)
    d2, c2 = _roll(d_ext, -2, 1), _roll(c_ext, -2, 1)
    md, mv = _combine(d_ext, c_ext, d1, c1, False)
    md, mv = _combine(md, mv, d2, c2, False)
    xd, xv = _combine(d_ext, c_ext, d1, c1, True)
    xd, xv = _combine(xd, xv, d2, c2, True)

    mdf, mvf = _combine(md[0:br], mv[0:br], md[1:1 + br], mv[1:1 + br], False)
    mdf, mvf = _combine(mdf, mvf, md[2:2 + br], mv[2:2 + br], False)
    xdf, xvf = _combine(xd[0:br], xv[0:br], xd[1:1 + br], xv[1:1 + br], True)
    xdf, xvf = _combine(xdf, xvf, xd[2:2 + br], xv[2:2 + br], True)

    out = mvf + xvf

    out = jnp.where(even_c, out, _roll(out, 1, 1))
    out = jnp.where(even_r, out, _roll(out, 1, 0))

    row_g = row_rel + row0
    out = jnp.where(col == w - 2, _roll(out, 2, 1), out)
    out = jnp.where(row_g == h - 2, _roll(out, 2, 0), out)
    out = jnp.where((row_g == h - 1) | (col == w - 1), 2.0 * c_ext[0:br], out)
    o_ref[...] = out


def _dc_kernel(a_ref, p_ref, n_ref, ah_ref, ph_ref, nh_ref, po_ref, no_ref,
               *, h, w, br):
    b = pl.program_id(0)
    row0 = b * br
    a_ext = jnp.concatenate([a_ref[...], ah_ref[...]], axis=0)
    p_ext = jnp.concatenate([p_ref[...], ph_ref[...]], axis=0)
    n_ext = jnp.concatenate([n_ref[...], nh_ref[...]], axis=0)
    _map_body(a_ext, p_ext, po_ref, row0, h, w)
    _map_body(a_ext, n_ext, no_ref, row0, h, w)


_SC_NC, _SC_NS, _SC_L = 2, 16, 16
_NW = _SC_NC * _SC_NS
_CH = 4


def _sc_body(a_hbm, p_hbm, n_hbm, pos_hbm, neg_hbm, a_v, p_v, n_v,
             pos_v, neg_v, in_sem, out_sem, *, h, w, row0):
    wid = lax.axis_index("s") * _SC_NC + lax.axis_index("c")
    nwr_tile = ((h - row0) // 2) // _NW
    nchunks = nwr_tile // _CH
    ngroups = (w // 2) // _SC_L
    iota = lax.iota(jnp.int32, _SC_L)
    lane_last = iota == (_SC_L - 1)

    out_cps = []
    for chunk in range(nchunks):
        i0 = row0 // 2 + wid * nwr_tile + chunk * _CH
        start = 2 * i0
        halo = jnp.minimum(start + 8, h - 8)
        in_cps = []
        for hbm, vbuf in ((a_hbm, a_v), (p_hbm, p_v), (n_hbm, n_v)):
            for r in range(8):
                in_cps.append(pltpu.async_copy(
                    hbm.at[start + r], vbuf.at[pl.ds(r * w, w)], in_sem))
            in_cps.append(pltpu.async_copy(
                hbm.at[halo], vbuf.at[pl.ds(8 * w, w)], in_sem))
        for cp in out_cps:
            cp.wait()
        out_cps = []
        for cp in in_cps:
            cp.wait()

        for ir in range(_CH):
            iw = i0 + ir
            base = jnp.minimum(2 * iw, h - 4) - start

            def group_body(g, carry, base=base, ir=ir):
                jv = g * _SC_L + iota
                cb = jnp.minimum(2 * jv, w - 4)
                pdm = pvm = pdx = pvx = None
                ndm = nvm = ndx = nvx = None
                for k in range(3):
                    rbase = jnp.broadcast_to((base + k) * w, (_SC_L,))
                    for l in range(3):
                        cvec = rbase + cb + l
                        av = plsc.load_gather(a_v, [cvec])
                        pv = plsc.load_gather(p_v, [cvec])
                        nv = plsc.load_gather(n_v, [cvec])
                        dp = jnp.abs(av - pv)
                        dn = jnp.abs(av - nv)
                        if pdm is None:
                            pdm = pdx = dp
                            pvm = pvx = pv
                            ndm = ndx = dn
                            nvm = nvx = nv
                        else:
                            m = dp < pdm
                            pdm = jnp.where(m, dp, pdm)
                            pvm = jnp.where(m, pv, pvm)
                            m = dp > pdx
                            pdx = jnp.where(m, dp, pdx)
                            pvx = jnp.where(m, pv, pvx)
                            m = dn < ndm
                            ndm = jnp.where(m, dn, ndm)
                            nvm = jnp.where(m, nv, nvm)
                            m = dn > ndx
                            ndx = jnp.where(m, dn, ndx)
                            nvx = jnp.where(m, nv, nvx)
                vp = pvm + pvx
                vn = nvm + nvx
                oc0 = 2 * jv + (2 * ir) * w
                oc1 = oc0 + 1
                oc2 = 2 * jv + (2 * ir + 1) * w
                oc3 = oc2 + 1
                plsc.store_scatter(pos_v, [oc0], vp)
                plsc.store_scatter(pos_v, [oc1], vp)
                plsc.store_scatter(pos_v, [oc2], vp)
                plsc.store_scatter(pos_v, [oc3], vp)
                plsc.store_scatter(neg_v, [oc0], vn)
                plsc.store_scatter(neg_v, [oc1], vn)
                plsc.store_scatter(neg_v, [oc2], vn)
                plsc.store_scatter(neg_v, [oc3], vn)
                return carry

            lax.fori_loop(0, ngroups, group_body, 0)

        for orel in range(2 * _CH):
            tail = pl.ds(orel * w + w - _SC_L, _SC_L)
            for c_v, o_v in ((p_v, pos_v), (n_v, neg_v)):
                o_v[tail] = jnp.where(lane_last, 2.0 * c_v[tail], o_v[tail])

        @pl.when(i0 + _CH == h // 2)
        def _fix_last_row():
            for cc in range(w // _SC_L):
                osl = pl.ds((2 * _CH - 1) * w + cc * _SC_L, _SC_L)
                csl = pl.ds(7 * w + cc * _SC_L, _SC_L)
                pos_v[osl] = 2.0 * p_v[csl]
                neg_v[osl] = 2.0 * n_v[csl]

        obase = 2 * i0 - row0
        for vbuf, hbm in ((pos_v, pos_hbm), (neg_v, neg_hbm)):
            for r in range(2 * _CH):
                out_cps.append(pltpu.async_copy(
                    vbuf.at[pl.ds(r * w, w)], hbm.at[obase + r], out_sem))
    for cp in out_cps:
        cp.wait()


def _sc_kernel(anchor, positive, negative, row0=0):
    h, w = anchor.shape
    mesh = plsc.VectorSubcoreMesh(core_axis_name="c", subcore_axis_name="s",
                                  num_cores=_SC_NC, num_subcores=_SC_NS)
    body = functools.partial(_sc_body, h=h, w=w, row0=row0)
    orows = h - row0
    pos, neg = pl.kernel(
        body,
        out_type=(jax.ShapeDtypeStruct((orows, w), jnp.float32),
                  jax.ShapeDtypeStruct((orows, w), jnp.float32)),
        mesh=mesh,
        scratch_types=[
            pltpu.VMEM((9 * w,), jnp.float32),
            pltpu.VMEM((9 * w,), jnp.float32),
            pltpu.VMEM((9 * w,), jnp.float32),
            pltpu.VMEM((2 * _CH * w,), jnp.float32),
            pltpu.VMEM((2 * _CH * w,), jnp.float32),
            pltpu.SemaphoreType.DMA,
            pltpu.SemaphoreType.DMA,
        ],
        compiler_params=pltpu.CompilerParams(needs_layout_passes=False),
    )(anchor, positive, negative)
    return pos, neg


def kernel(anchor, positive, negative):
    tc_pos, tc_neg = _tc_kernel(anchor, positive, negative, out_rows=_TC_ROWS)
    sc_pos, sc_neg = _sc_kernel(anchor, positive, negative, row0=_TC_ROWS)
    return _splice(tc_pos, tc_neg, sc_pos, sc_neg, _TC_ROWS)


def _splice_body(tcp_any, tcn_any, scp_ref, scn_ref, po_ref, no_ref):
    del tcp_any, tcn_any
    po_ref[...] = scp_ref[...]
    no_ref[...] = scn_ref[...]


def _splice(tc_pos, tc_neg, sc_pos, sc_neg, row0):
    h, w = tc_pos.shape
    rows = sc_pos.shape[0]
    br = 128
    nb = rows // br
    ob = row0 // br
    return pl.pallas_call(
        _splice_body,
        grid=(nb,),
        in_specs=[pl.BlockSpec(memory_space=pl.ANY),
                  pl.BlockSpec(memory_space=pl.ANY),
                  pl.BlockSpec((br, w), lambda b: (b, 0)),
                  pl.BlockSpec((br, w), lambda b: (b, 0))],
        out_specs=[pl.BlockSpec((br, w), lambda b: (ob + b, 0))] * 2,
        out_shape=[jax.ShapeDtypeStruct((h, w), jnp.float32)] * 2,
        input_output_aliases={0: 0, 1: 1},
        compiler_params=pltpu.CompilerParams(
            dimension_semantics=("arbitrary",),
        ),
    )(tc_pos, tc_neg, sc_pos, sc_neg)


_TC_ROWS = 1024


def _tc_kernel(anchor, positive, negative, out_rows=None):
    h, w = anchor.shape
    br = min(_BR, h)
    if out_rows is None:
        out_rows = h
    nb = out_rows // br
    halo_blocks = h // _HALO

    def main_spec():
        return pl.BlockSpec((br, w), lambda b: (b, 0))

    def halo_spec():
        return pl.BlockSpec(
            (_HALO, w),
            lambda b: (jnp.minimum((b + 1) * (br // _HALO), halo_blocks - 1), 0),
        )

    body = functools.partial(_dc_kernel, h=h, w=w, br=br)
    pos, neg = pl.pallas_call(
        body,
        grid=(nb,),
        in_specs=[main_spec(), main_spec(), main_spec(),
                  halo_spec(), halo_spec(), halo_spec()],
        out_specs=[pl.BlockSpec((br, w), lambda b: (b, 0))] * 2,
        out_shape=[jax.ShapeDtypeStruct((h, w), jnp.float32)] * 2,
        compiler_params=pltpu.CompilerParams(
            dimension_semantics=("arbitrary",),
        ),
    )(anchor, positive, negative, anchor, positive, negative)
    return (pos, neg)
